# baseline (speedup 1.0000x reference)
import functools
import math

import jax
import jax.numpy as jnp
import numpy as np
from jax import lax
from jax.experimental import pallas as pl
from jax.experimental.pallas import tpu as pltpu
from jax.experimental.pallas import tpu_sc as plsc

B = 2
L_IN = 2048
INPUT_DIM = 256
D_MODEL = 768
NHEAD = 12
D_K = 64
N_LAYERS = 2
DFF = 3072
DEC_H = 768
PRED_H = 100
FACTOR = 3
BH = B * NHEAD
LT = 512
UPAD = 32
DP = 128


def _np_rotl(x, d):
    return ((x << np.uint32(d)) | (x >> np.uint32(32 - d))).astype(np.uint32)


def _np_threefry2x32(keypair, count):
    odd = count.size % 2
    arr = count.ravel()
    if odd:
        arr = np.concatenate([arr, np.zeros(1, np.uint32)])
    x = [arr[:arr.size // 2].copy(), arr[arr.size // 2:].copy()]
    ks0 = np.uint32(keypair[0])
    ks1 = np.uint32(keypair[1])
    ks2 = np.uint32(ks0 ^ ks1 ^ np.uint32(0x1BD11BDA))
    rot = [(13, 15, 26, 6), (17, 29, 16, 24)]

    def rounds(x, rs):
        for r in rs:
            x[0] = (x[0] + x[1]).astype(np.uint32)
            x[1] = _np_rotl(x[1], r)
            x[1] = (x[1] ^ x[0]).astype(np.uint32)
        return x

    x[0] = (x[0] + ks0).astype(np.uint32)
    x[1] = (x[1] + ks1).astype(np.uint32)
    for i, (inj0, inj1) in enumerate([(ks1, ks2), (ks2, ks0), (ks0, ks1),
                                      (ks1, ks2), (ks2, ks0)]):
        x = rounds(x, rot[i % 2])
        x[0] = (x[0] + inj0).astype(np.uint32)
        x[1] = (x[1] + inj1 + np.uint32(i + 1)).astype(np.uint32)
    out = np.concatenate(x)
    return out[:count.size] if odd else out


def _np_tf_pair(keypair, c1, c2):
    n = c1.size
    out = _np_threefry2x32(keypair, np.concatenate([c1, c2]).astype(np.uint32))
    return out[:n], out[n:]


def _np_seed(s):
    return np.array([(s >> 32) & 0xFFFFFFFF, s & 0xFFFFFFFF], np.uint32)


def _np_randint(keypair, shape, lc):
    i2 = np.arange(2, dtype=np.uint64)
    b1, b2 = _np_tf_pair(keypair, (i2 >> np.uint64(32)).astype(np.uint32),
                         i2.astype(np.uint32))
    keys = np.stack([b1, b2], axis=1)

    def bits(kp):
        i = np.arange(int(np.prod(shape)), dtype=np.uint64)
        h, l2 = _np_tf_pair(kp, (i >> np.uint64(32)).astype(np.uint32),
                            i.astype(np.uint32))
        return (h ^ l2).reshape(shape)

    higher, lower = bits(keys[0]), bits(keys[1])
    span = np.uint32(lc)
    mult = np.uint32((((2 ** 16) % lc) ** 2) % lc)
    off = ((higher % span) * mult + (lower % span)) % span
    return off.astype(np.int64)


@functools.lru_cache(maxsize=None)
def _sample_tables(layer, lc, u):
    key = _np_threefry2x32(_np_seed(1234), _np_seed(layer))
    idx = _np_randint(key, (lc, u), lc)
    cnt = np.zeros((lc, lc), np.float32)
    np.add.at(cnt, (np.arange(lc)[:, None], idx), 1.0)
    msk = np.where(cnt > 0, 0.0, -np.inf).astype(np.float32)
    return jnp.asarray(cnt), jnp.asarray(msk)


def _pe_table(max_len, d_model):
    position = np.arange(max_len, dtype=np.float32)[:, None]
    div_term = np.exp(np.arange(0, d_model, 2, dtype=np.float32) * (-math.log(10000.0) / d_model))
    pe = np.zeros((max_len, d_model), dtype=np.float32)
    pe[:, 0::2] = np.sin(position * div_term)
    pe[:, 1::2] = np.cos(position * div_term)
    return jnp.asarray(pe)


def _ln(x, g, b):
    m = jnp.mean(x, axis=1, keepdims=True)
    c = x - m
    v = jnp.mean(c * c, axis=1, keepdims=True)
    return c / jnp.sqrt(v + 1e-5) * g + b


def _embed_body(x_ref, w_ref, b_ref, pe_ref, o_ref):
    o_ref[...] = (jnp.dot(x_ref[...], w_ref[...], preferred_element_type=jnp.float32)
                  + b_ref[...] + pe_ref[...])


def _embed(x2, w, b, pe):
    n = x2.shape[0]
    nt = n // LT
    pt = pe.shape[0] // LT
    return pl.pallas_call(
        _embed_body,
        grid=(nt,),
        in_specs=[pl.BlockSpec((LT, INPUT_DIM), lambda i: (i, 0)),
                  pl.BlockSpec((INPUT_DIM, D_MODEL), lambda i: (0, 0)),
                  pl.BlockSpec((1, D_MODEL), lambda i: (0, 0)),
                  pl.BlockSpec((LT, D_MODEL), lambda i: (i % pt, 0))],
        out_specs=pl.BlockSpec((LT, D_MODEL), lambda i: (i, 0)),
        out_shape=jax.ShapeDtypeStruct((n, D_MODEL), jnp.float32),
    )(x2, w, b.reshape(1, -1), pe)


def _qkv_body(e_ref, wq_ref, wk_ref, wv_ref, bq_ref, bk_ref, bv_ref,
              q_ref, k_ref, v_ref):
    x = e_ref[...]
    yq = jnp.dot(x, wq_ref[...], preferred_element_type=jnp.float32) + bq_ref[...]
    yk = jnp.dot(x, wk_ref[...], preferred_element_type=jnp.float32) + bk_ref[...]
    yv = jnp.dot(x, wv_ref[...], preferred_element_type=jnp.float32) + bv_ref[...]
    for h in range(NHEAD):
        q_ref[h, :, :D_K] = yq[:, h * D_K:(h + 1) * D_K]
        k_ref[h, :, :D_K] = yk[:, h * D_K:(h + 1) * D_K]
        v_ref[h, :, :D_K] = yv[:, h * D_K:(h + 1) * D_K]


def _qkv(e, wq, wk, wv, bq, bk, bv, lc):
    nt = lc // LT
    espec = pl.BlockSpec((LT, D_MODEL), lambda i: (i, 0))
    wspec = pl.BlockSpec((D_MODEL, D_MODEL), lambda i: (0, 0))
    bspec = pl.BlockSpec((1, D_MODEL), lambda i: (0, 0))
    ospec = pl.BlockSpec((NHEAD, LT, DP), lambda i: (i // nt, i % nt, 0))
    out = jax.ShapeDtypeStruct((BH, lc, DP), jnp.float32)
    return pl.pallas_call(
        _qkv_body,
        grid=(B * nt,),
        in_specs=[espec, wspec, wspec, wspec, bspec, bspec, bspec],
        out_specs=[ospec, ospec, ospec],
        out_shape=[out, out, out],
    )(e, wq, wk, wv, bq, bk, bv)


def _m_scores(q, k, cnt, msk, lc):
    nt = lc // LT

    def body(q_ref, k_ref, c_ref, s_ref, m_ref):
        q64 = q_ref[0][:, :D_K]
        k64 = k_ref[0][:, :D_K]
        s = lax.dot_general(q64, k64, (((1,), (1,)), ((), ())),
                            preferred_element_type=jnp.float32)
        mmax = jnp.max(s + s_ref[...], axis=1, keepdims=True)
        msum = jnp.sum(s * c_ref[...], axis=1, keepdims=True)
        m_ref[0, 0] = (mmax - msum * (1.0 / lc)).reshape(1, LT)

    m4 = pl.pallas_call(
        body,
        grid=(nt, BH),
        in_specs=[pl.BlockSpec((1, LT, DP), lambda t, bh: (bh, t, 0)),
                  pl.BlockSpec((1, lc, DP), lambda t, bh: (bh, 0, 0)),
                  pl.BlockSpec((LT, lc), lambda t, bh: (t, 0)),
                  pl.BlockSpec((LT, lc), lambda t, bh: (t, 0))],
        out_specs=pl.BlockSpec((1, 1, 1, LT), lambda t, bh: (bh, t, 0, 0)),
        out_shape=jax.ShapeDtypeStruct((BH, nt, 1, LT), jnp.float32),
    )(q, k, cnt, msk)
    return m4.reshape(BH, lc)


def _topk(m, lc, u):
    def body(m_ref, o_ref):
        mw = m_ref[...]
        iota = lax.broadcasted_iota(jnp.int32, (BH, lc), 1)
        cols = []
        for _ in range(u):
            mx = jnp.max(mw, axis=1, keepdims=True)
            am = jnp.min(jnp.where(mw == mx, iota, lc), axis=1, keepdims=True)
            cols.append(am)
            mw = jnp.where(iota == am, -jnp.inf, mw)
        cols.extend([cols[0]] * (UPAD - u))
        o_ref[...] = jnp.concatenate(cols, axis=1)

    return pl.pallas_call(
        body,
        in_specs=[pl.BlockSpec((BH, lc), lambda: (0, 0))],
        out_specs=pl.BlockSpec((BH, UPAD), lambda: (0, 0)),
        out_shape=jax.ShapeDtypeStruct((BH, UPAD), jnp.int32),
    )(m)


def _sc_gather(q, mtop_flat, lc):
    mesh = plsc.VectorSubcoreMesh(core_axis_name="c", subcore_axis_name="s")

    @functools.partial(
        pl.kernel,
        out_type=jax.ShapeDtypeStruct((BH, UPAD, DP), jnp.float32),
        mesh=mesh,
        scratch_types=[pltpu.VMEM((UPAD,), jnp.int32),
                       pltpu.VMEM((UPAD, DP), jnp.float32),
                       pltpu.SemaphoreType.DMA],
    )
    def k(q_hbm, mtop_hbm, out_hbm, idx_v, rows_v, sem):
        wid = lax.axis_index("s") * 2 + lax.axis_index("c")

        @pl.when(wid < BH)
        def _():
            pltpu.sync_copy(mtop_hbm.at[pl.ds(wid * UPAD, UPAD)], idx_v)
            pltpu.async_copy(q_hbm.at[wid].at[idx_v], rows_v, sem).wait()
            pltpu.sync_copy(rows_v, out_hbm.at[wid])

    return k(q, mtop_flat)


def _attn_red(qred, k, v, lc):
    def body(q_ref, k_ref, v_ref, c_ref, vm_ref):
        s = lax.dot_general(q_ref[0][:, :D_K], k_ref[0][:, :D_K],
                            (((1,), (1,)), ((), ())),
                            preferred_element_type=jnp.float32) * (1.0 / math.sqrt(D_K))
        mx = jnp.max(s, axis=1, keepdims=True)
        e = jnp.exp(s - mx)
        attn = e / jnp.sum(e, axis=1, keepdims=True)
        c_ref[0] = jnp.dot(attn, v_ref[0], preferred_element_type=jnp.float32)
        vm_ref[0] = jnp.broadcast_to(jnp.mean(v_ref[0], axis=0, keepdims=True), (128, DP))

    return pl.pallas_call(
        body,
        grid=(BH,),
        in_specs=[pl.BlockSpec((1, UPAD, DP), lambda i: (i, 0, 0)),
                  pl.BlockSpec((1, lc, DP), lambda i: (i, 0, 0)),
                  pl.BlockSpec((1, lc, DP), lambda i: (i, 0, 0))],
        out_specs=[pl.BlockSpec((1, UPAD, DP), lambda i: (i, 0, 0)),
                   pl.BlockSpec((1, 128, DP), lambda i: (i, 0, 0))],
        out_shape=[jax.ShapeDtypeStruct((BH, UPAD, DP), jnp.float32),
                   jax.ShapeDtypeStruct((BH, 128, DP), jnp.float32)],
    )(qred, k, v)


def _sc_context(ctx, mtop_flat, vmrep, lc):
    mesh = plsc.VectorSubcoreMesh(core_axis_name="c", subcore_axis_name="s")
    nrep = 128

    @functools.partial(
        pl.kernel,
        out_type=jax.ShapeDtypeStruct((BH, lc, DP), jnp.float32),
        mesh=mesh,
        scratch_types=[pltpu.VMEM((UPAD,), jnp.int32),
                       pltpu.VMEM((UPAD, DP), jnp.float32),
                       pltpu.VMEM((nrep, DP), jnp.float32),
                       pltpu.SemaphoreType.DMA],
    )
    def k(ctx_hbm, mtop_hbm, vm_hbm, out_hbm, idx_v, rows_v, rep_v, sem):
        wid = lax.axis_index("s") * 2 + lax.axis_index("c")

        @pl.when(wid < BH)
        def _():
            pltpu.sync_copy(vm_hbm.at[wid], rep_v)
            for c0 in range(lc // nrep):
                pltpu.sync_copy(rep_v, out_hbm.at[wid, pl.ds(c0 * nrep, nrep)])
            pltpu.sync_copy(mtop_hbm.at[pl.ds(wid * UPAD, UPAD)], idx_v)
            pltpu.sync_copy(ctx_hbm.at[wid], rows_v)
            pltpu.async_copy(rows_v, out_hbm.at[wid].at[idx_v], sem).wait()

    return k(ctx, mtop_flat, vmrep)


def _wo_ln1(ctx, src, wo_r, bo, g, bln, lc):
    nt = lc // LT

    def body(c_ref, s_ref, w_ref, bo_ref, g_ref, b_ref, o_ref):
        acc = jnp.zeros((LT, D_MODEL), jnp.float32)
        for j in range(NHEAD // 4):
            z = jnp.concatenate([c_ref[4 * j + i][:, :D_K] for i in range(4)],
                                axis=1)
            acc = acc + jnp.dot(z, w_ref[j], preferred_element_type=jnp.float32)
        x = acc + bo_ref[...] + s_ref[...]
        o_ref[...] = _ln(x, g_ref[...], b_ref[...])

    return pl.pallas_call(
        body,
        grid=(B, nt),
        in_specs=[pl.BlockSpec((NHEAD, LT, DP), lambda b, t: (b, t, 0)),
                  pl.BlockSpec((LT, D_MODEL), lambda b, t: (b * nt + t, 0)),
                  pl.BlockSpec((NHEAD // 4, 4 * D_K, D_MODEL), lambda b, t: (0, 0, 0)),
                  pl.BlockSpec((1, D_MODEL), lambda b, t: (0, 0)),
                  pl.BlockSpec((1, D_MODEL), lambda b, t: (0, 0)),
                  pl.BlockSpec((1, D_MODEL), lambda b, t: (0, 0))],
        out_specs=pl.BlockSpec((LT, D_MODEL), lambda b, t: (b * nt + t, 0)),
        out_shape=jax.ShapeDtypeStruct((B * lc, D_MODEL), jnp.float32),
    )(ctx, src, wo_r, bo.reshape(1, -1), g.reshape(1, -1), bln.reshape(1, -1))


def _ffn_ln2(src, w1, b1, w2, b2, g, bln, lowp=False):
    n = src.shape[0]
    nt = n // LT

    def body(s_ref, w1_ref, b1_ref, w2_ref, b2_ref, g_ref, b_ref, o_ref):
        x = s_ref[...]
        if lowp:
            t = jnp.dot(x.astype(jnp.bfloat16), w1_ref[...].astype(jnp.bfloat16),
                        preferred_element_type=jnp.float32) + b1_ref[...]
            ge = 0.5 * t * (1.0 + lax.erf(t * (1.0 / math.sqrt(2.0))))
            y = jnp.dot(ge.astype(jnp.bfloat16), w2_ref[...].astype(jnp.bfloat16),
                        preferred_element_type=jnp.float32) + b2_ref[...] + x
        else:
            t = jnp.dot(x, w1_ref[...], preferred_element_type=jnp.float32) + b1_ref[...]
            ge = 0.5 * t * (1.0 + lax.erf(t * (1.0 / math.sqrt(2.0))))
            y = jnp.dot(ge, w2_ref[...], preferred_element_type=jnp.float32) + b2_ref[...] + x
        o_ref[...] = _ln(y, g_ref[...], b_ref[...])

    return pl.pallas_call(
        body,
        grid=(nt,),
        in_specs=[pl.BlockSpec((LT, D_MODEL), lambda i: (i, 0)),
                  pl.BlockSpec((D_MODEL, DFF), lambda i: (0, 0)),
                  pl.BlockSpec((1, DFF), lambda i: (0, 0)),
                  pl.BlockSpec((DFF, D_MODEL), lambda i: (0, 0)),
                  pl.BlockSpec((1, D_MODEL), lambda i: (0, 0)),
                  pl.BlockSpec((1, D_MODEL), lambda i: (0, 0)),
                  pl.BlockSpec((1, D_MODEL), lambda i: (0, 0))],
        out_specs=pl.BlockSpec((LT, D_MODEL), lambda i: (i, 0)),
        out_shape=jax.ShapeDtypeStruct((n, D_MODEL), jnp.float32),
    )(src, w1, b1.reshape(1, -1), w2, b2.reshape(1, -1),
      g.reshape(1, -1), bln.reshape(1, -1))


def _conv_pool(src, w0, w1, w2, bc, scale, bnb, lc):
    def body(s_ref, w0_ref, w1_ref, w2_ref, bc_ref, sc_ref, bb_ref, o_ref):
        x = s_ref[...]
        xm = jnp.concatenate([x[lc - 1:], x[:lc - 1]], axis=0)
        xp = jnp.concatenate([x[1:], x[:1]], axis=0)
        y = (jnp.dot(xm, w0_ref[...], preferred_element_type=jnp.float32)
             + jnp.dot(x, w1_ref[...], preferred_element_type=jnp.float32)
             + jnp.dot(xp, w2_ref[...], preferred_element_type=jnp.float32)
             + bc_ref[...])
        y = y * sc_ref[...] + bb_ref[...]
        y = jnp.where(y > 0, y, jnp.exp(y) - 1.0)
        yr = y.reshape(lc // 2, 2, D_MODEL)
        a = yr[:, 0, :]
        bcol = yr[:, 1, :]
        bm = jnp.concatenate(
            [jnp.full((1, D_MODEL), -jnp.inf, jnp.float32), bcol[:-1]], axis=0)
        o_ref[...] = jnp.maximum(jnp.maximum(a, bcol), bm)

    return pl.pallas_call(
        body,
        grid=(B,),
        in_specs=[pl.BlockSpec((lc, D_MODEL), lambda b: (b, 0)),
                  pl.BlockSpec((D_MODEL, D_MODEL), lambda b: (0, 0)),
                  pl.BlockSpec((D_MODEL, D_MODEL), lambda b: (0, 0)),
                  pl.BlockSpec((D_MODEL, D_MODEL), lambda b: (0, 0)),
                  pl.BlockSpec((1, D_MODEL), lambda b: (0, 0)),
                  pl.BlockSpec((1, D_MODEL), lambda b: (0, 0)),
                  pl.BlockSpec((1, D_MODEL), lambda b: (0, 0))],
        out_specs=pl.BlockSpec((lc // 2, D_MODEL), lambda b: (b, 0)),
        out_shape=jax.ShapeDtypeStruct((B * (lc // 2), D_MODEL), jnp.float32),
    )(src, w0, w1, w2, bc.reshape(1, -1), scale.reshape(1, -1), bnb.reshape(1, -1))


def _gru(dec8, wi, wh, bi, bhb):
    def body(d_ref, wi_ref, wh_ref, bi_ref, bh_ref, hs_ref):
        gi = jnp.dot(d_ref[...], wi_ref[...], preferred_element_type=jnp.float32) + bi_ref[...]
        whb = wh_ref[...].astype(jnp.bfloat16)

        def step(t, h):
            gh = jnp.dot(h.astype(jnp.bfloat16), whb,
                         preferred_element_type=jnp.float32) + bh_ref[...]
            r = jax.nn.sigmoid(gi[:, :DEC_H] + gh[:, :DEC_H])
            z = jax.nn.sigmoid(gi[:, DEC_H:2 * DEC_H] + gh[:, DEC_H:2 * DEC_H])
            nn = jnp.tanh(gi[:, 2 * DEC_H:] + r * gh[:, 2 * DEC_H:])
            hn = (1.0 - z) * nn + z * h
            hs_ref[pl.ds(t, 1)] = hn[None]
            return hn

        lax.fori_loop(0, PRED_H, step, jnp.zeros((8, DEC_H), jnp.float32))

    return pl.pallas_call(
        body,
        in_specs=[pl.BlockSpec((8, D_MODEL), lambda: (0, 0)),
                  pl.BlockSpec((D_MODEL, 3 * DEC_H), lambda: (0, 0)),
                  pl.BlockSpec((DEC_H, 3 * DEC_H), lambda: (0, 0)),
                  pl.BlockSpec((1, 3 * DEC_H), lambda: (0, 0)),
                  pl.BlockSpec((1, 3 * DEC_H), lambda: (0, 0))],
        out_specs=pl.BlockSpec((PRED_H, 8, DEC_H), lambda: (0, 0, 0)),
        out_shape=jax.ShapeDtypeStruct((PRED_H, 8, DEC_H), jnp.float32),
    )(dec8, wi, wh, bi.reshape(1, -1), bhb.reshape(1, -1))


def _outproj(hs_flat, w, b):
    n = hs_flat.shape[0]

    def body(h_ref, w_ref, b_ref, o_ref):
        o_ref[...] = jax.nn.sigmoid(
            jnp.dot(h_ref[...], w_ref[...], preferred_element_type=jnp.float32) + b_ref[...])

    return pl.pallas_call(
        body,
        in_specs=[pl.BlockSpec((n, DEC_H), lambda: (0, 0)),
                  pl.BlockSpec((DEC_H, 1), lambda: (0, 0)),
                  pl.BlockSpec((1, 1), lambda: (0, 0))],
        out_specs=pl.BlockSpec((n, 1), lambda: (0, 0)),
        out_shape=jax.ShapeDtypeStruct((n, 1), jnp.float32),
    )(hs_flat, w, b.reshape(1, 1))


def kernel(x, params):
    p = params
    x2 = x.reshape(B * L_IN, INPUT_DIM)
    pe = _pe_table(L_IN, D_MODEL)
    h = _embed(x2, p['emb_W'], p['emb_b'], pe)

    lc = L_IN
    for l in range(N_LAYERS):
        lp = p['layers'][l]
        u = min(FACTOR * int(np.ceil(np.log(lc + 1))), lc)
        cnt, msk = _sample_tables(l, lc, u)
        q, k, v = _qkv(h, lp['Wq'], lp['Wk'], lp['Wv'],
                       lp['bq'].reshape(1, -1), lp['bk'].reshape(1, -1),
                       lp['bv'].reshape(1, -1), lc)
        m = _m_scores(q, k, cnt, msk, lc)
        mtop = _topk(m, lc, u).reshape(BH * UPAD)
        qred = _sc_gather(q, mtop, lc)
        ctx, vmrep = _attn_red(qred, k, v, lc)
        ctxfull = _sc_context(ctx, mtop, vmrep, lc)
        wo_r = lp['Wo'].reshape(NHEAD // 4, 4 * D_K, D_MODEL)
        h = _wo_ln1(ctxfull, h, wo_r, lp['bo'], lp['ln1_g'], lp['ln1_b'], lc)
        h = _ffn_ln2(h, lp['W1'], lp['b1'], lp['W2'], lp['b2'],
                     lp['ln2_g'], lp['ln2_b'], lowp=(l == N_LAYERS - 1))
        if l < N_LAYERS - 1:
            cp = p['convs'][l]
            w0 = cp['w'][:, :, 0].T
            w1 = cp['w'][:, :, 1].T
            w2 = cp['w'][:, :, 2].T
            scale = cp['bn_g'] / jnp.sqrt(jnp.float32(1.0) + 1e-5)
            h = _conv_pool(h, w0, w1, w2, cp['b'], scale, cp['bn_b'], lc)
            lc = lc // 2

    dec_rows = jnp.stack([h[lc - 1], h[2 * lc - 1]], axis=0)
    dec8 = jnp.zeros((8, D_MODEL), jnp.float32).at[:B].set(dec_rows)
    hs = _gru(dec8, p['gru_Wi'], p['gru_Wh'], p['gru_bi'], p['gru_bh'])
    o = _outproj(hs.reshape(PRED_H * 8, DEC_H), p['out_W'], p['out_b'])
    return o.reshape(PRED_H, 8)[:, :B].T


# speedup vs baseline: 1.0007x; 1.0007x over previous
import functools
import math

import jax
import jax.numpy as jnp
import numpy as np
from jax import lax
from jax.experimental import pallas as pl
from jax.experimental.pallas import tpu as pltpu
from jax.experimental.pallas import tpu_sc as plsc

B = 2
L_IN = 2048
INPUT_DIM = 256
D_MODEL = 768
NHEAD = 12
D_K = 64
N_LAYERS = 2
DFF = 3072
DEC_H = 768
PRED_H = 100
FACTOR = 3
BH = B * NHEAD
LT = 512
UPAD = 32
DP = 128


def _np_rotl(x, d):
    return ((x << np.uint32(d)) | (x >> np.uint32(32 - d))).astype(np.uint32)


def _np_threefry2x32(keypair, count):
    odd = count.size % 2
    arr = count.ravel()
    if odd:
        arr = np.concatenate([arr, np.zeros(1, np.uint32)])
    x = [arr[:arr.size // 2].copy(), arr[arr.size // 2:].copy()]
    ks0 = np.uint32(keypair[0])
    ks1 = np.uint32(keypair[1])
    ks2 = np.uint32(ks0 ^ ks1 ^ np.uint32(0x1BD11BDA))
    rot = [(13, 15, 26, 6), (17, 29, 16, 24)]

    def rounds(x, rs):
        for r in rs:
            x[0] = (x[0] + x[1]).astype(np.uint32)
            x[1] = _np_rotl(x[1], r)
            x[1] = (x[1] ^ x[0]).astype(np.uint32)
        return x

    x[0] = (x[0] + ks0).astype(np.uint32)
    x[1] = (x[1] + ks1).astype(np.uint32)
    for i, (inj0, inj1) in enumerate([(ks1, ks2), (ks2, ks0), (ks0, ks1),
                                      (ks1, ks2), (ks2, ks0)]):
        x = rounds(x, rot[i % 2])
        x[0] = (x[0] + inj0).astype(np.uint32)
        x[1] = (x[1] + inj1 + np.uint32(i + 1)).astype(np.uint32)
    out = np.concatenate(x)
    return out[:count.size] if odd else out


def _np_tf_pair(keypair, c1, c2):
    n = c1.size
    out = _np_threefry2x32(keypair, np.concatenate([c1, c2]).astype(np.uint32))
    return out[:n], out[n:]


def _np_seed(s):
    return np.array([(s >> 32) & 0xFFFFFFFF, s & 0xFFFFFFFF], np.uint32)


def _np_randint(keypair, shape, lc):
    i2 = np.arange(2, dtype=np.uint64)
    b1, b2 = _np_tf_pair(keypair, (i2 >> np.uint64(32)).astype(np.uint32),
                         i2.astype(np.uint32))
    keys = np.stack([b1, b2], axis=1)

    def bits(kp):
        i = np.arange(int(np.prod(shape)), dtype=np.uint64)
        h, l2 = _np_tf_pair(kp, (i >> np.uint64(32)).astype(np.uint32),
                            i.astype(np.uint32))
        return (h ^ l2).reshape(shape)

    higher, lower = bits(keys[0]), bits(keys[1])
    span = np.uint32(lc)
    mult = np.uint32((((2 ** 16) % lc) ** 2) % lc)
    off = ((higher % span) * mult + (lower % span)) % span
    return off.astype(np.int64)


@functools.lru_cache(maxsize=None)
def _sample_tables(layer, lc, u):
    key = _np_threefry2x32(_np_seed(1234), _np_seed(layer))
    idx = _np_randint(key, (lc, u), lc)
    cnt = np.zeros((lc, lc), np.float32)
    np.add.at(cnt, (np.arange(lc)[:, None], idx), 1.0)
    msk = np.where(cnt > 0, 0.0, -np.inf).astype(np.float32)
    return jnp.asarray(cnt), jnp.asarray(msk)


def _pe_table(max_len, d_model):
    position = np.arange(max_len, dtype=np.float32)[:, None]
    div_term = np.exp(np.arange(0, d_model, 2, dtype=np.float32) * (-math.log(10000.0) / d_model))
    pe = np.zeros((max_len, d_model), dtype=np.float32)
    pe[:, 0::2] = np.sin(position * div_term)
    pe[:, 1::2] = np.cos(position * div_term)
    return jnp.asarray(pe)


def _ln(x, g, b):
    m = jnp.mean(x, axis=1, keepdims=True)
    c = x - m
    v = jnp.mean(c * c, axis=1, keepdims=True)
    return c / jnp.sqrt(v + 1e-5) * g + b


def _embed_body(x_ref, w_ref, b_ref, pe_ref, o_ref):
    o_ref[...] = (jnp.dot(x_ref[...], w_ref[...], preferred_element_type=jnp.float32)
                  + b_ref[...] + pe_ref[...])


def _embed(x2, w, b, pe):
    n = x2.shape[0]
    nt = n // LT
    pt = pe.shape[0] // LT
    return pl.pallas_call(
        _embed_body,
        grid=(nt,),
        in_specs=[pl.BlockSpec((LT, INPUT_DIM), lambda i: (i, 0)),
                  pl.BlockSpec((INPUT_DIM, D_MODEL), lambda i: (0, 0)),
                  pl.BlockSpec((1, D_MODEL), lambda i: (0, 0)),
                  pl.BlockSpec((LT, D_MODEL), lambda i: (i % pt, 0))],
        out_specs=pl.BlockSpec((LT, D_MODEL), lambda i: (i, 0)),
        out_shape=jax.ShapeDtypeStruct((n, D_MODEL), jnp.float32),
    )(x2, w, b.reshape(1, -1), pe)


def _qkv_body(e_ref, wq_ref, wk_ref, wv_ref, bq_ref, bk_ref, bv_ref,
              q_ref, k_ref, v_ref):
    x = e_ref[...]
    yq = jnp.dot(x, wq_ref[...], preferred_element_type=jnp.float32) + bq_ref[...]
    yk = jnp.dot(x, wk_ref[...], preferred_element_type=jnp.float32) + bk_ref[...]
    yv = jnp.dot(x, wv_ref[...], preferred_element_type=jnp.float32) + bv_ref[...]
    for h in range(NHEAD):
        q_ref[h, :, :D_K] = yq[:, h * D_K:(h + 1) * D_K]
        k_ref[h, :, :D_K] = yk[:, h * D_K:(h + 1) * D_K]
        v_ref[h, :, :D_K] = yv[:, h * D_K:(h + 1) * D_K]


def _qkv(e, wq, wk, wv, bq, bk, bv, lc):
    nt = lc // LT
    espec = pl.BlockSpec((LT, D_MODEL), lambda i: (i, 0))
    wspec = pl.BlockSpec((D_MODEL, D_MODEL), lambda i: (0, 0))
    bspec = pl.BlockSpec((1, D_MODEL), lambda i: (0, 0))
    ospec = pl.BlockSpec((NHEAD, LT, DP), lambda i: (i // nt, i % nt, 0))
    out = jax.ShapeDtypeStruct((BH, lc, DP), jnp.float32)
    return pl.pallas_call(
        _qkv_body,
        grid=(B * nt,),
        in_specs=[espec, wspec, wspec, wspec, bspec, bspec, bspec],
        out_specs=[ospec, ospec, ospec],
        out_shape=[out, out, out],
    )(e, wq, wk, wv, bq, bk, bv)


def _m_scores(q, k, cnt, msk, lc):
    nt = lc // LT

    def body(q_ref, k_ref, c_ref, s_ref, m_ref):
        q64 = q_ref[0][:, :D_K]
        k64 = k_ref[0][:, :D_K]
        s = lax.dot_general(q64, k64, (((1,), (1,)), ((), ())),
                            preferred_element_type=jnp.float32)
        mmax = jnp.max(s + s_ref[...], axis=1, keepdims=True)
        msum = jnp.sum(s * c_ref[...], axis=1, keepdims=True)
        m_ref[0, 0] = (mmax - msum * (1.0 / lc)).reshape(1, LT)

    m4 = pl.pallas_call(
        body,
        grid=(nt, BH),
        in_specs=[pl.BlockSpec((1, LT, DP), lambda t, bh: (bh, t, 0)),
                  pl.BlockSpec((1, lc, DP), lambda t, bh: (bh, 0, 0)),
                  pl.BlockSpec((LT, lc), lambda t, bh: (t, 0)),
                  pl.BlockSpec((LT, lc), lambda t, bh: (t, 0))],
        out_specs=pl.BlockSpec((1, 1, 1, LT), lambda t, bh: (bh, t, 0, 0)),
        out_shape=jax.ShapeDtypeStruct((BH, nt, 1, LT), jnp.float32),
    )(q, k, cnt, msk)
    return m4.reshape(BH, lc)


def _topk(m, lc, u):
    def body(m_ref, o_ref):
        mw = m_ref[...]
        iota = lax.broadcasted_iota(jnp.int32, (BH, lc), 1)
        cols = []
        for _ in range(u):
            mx = jnp.max(mw, axis=1, keepdims=True)
            am = jnp.min(jnp.where(mw == mx, iota, lc), axis=1, keepdims=True)
            cols.append(am)
            mw = jnp.where(iota == am, -jnp.inf, mw)
        cols.extend([cols[0]] * (UPAD - u))
        o_ref[...] = jnp.concatenate(cols, axis=1)

    return pl.pallas_call(
        body,
        in_specs=[pl.BlockSpec((BH, lc), lambda: (0, 0))],
        out_specs=pl.BlockSpec((BH, UPAD), lambda: (0, 0)),
        out_shape=jax.ShapeDtypeStruct((BH, UPAD), jnp.int32),
    )(m)


def _sc_gather(q, mtop_flat, lc):
    mesh = plsc.VectorSubcoreMesh(core_axis_name="c", subcore_axis_name="s")

    @functools.partial(
        pl.kernel,
        out_type=jax.ShapeDtypeStruct((BH, UPAD, DP), jnp.float32),
        mesh=mesh,
        scratch_types=[pltpu.VMEM((UPAD,), jnp.int32),
                       pltpu.VMEM((UPAD, DP), jnp.float32),
                       pltpu.SemaphoreType.DMA],
    )
    def k(q_hbm, mtop_hbm, out_hbm, idx_v, rows_v, sem):
        wid = lax.axis_index("s") * 2 + lax.axis_index("c")

        @pl.when(wid < BH)
        def _():
            pltpu.sync_copy(mtop_hbm.at[pl.ds(wid * UPAD, UPAD)], idx_v)
            pltpu.async_copy(q_hbm.at[wid].at[idx_v], rows_v, sem).wait()
            pltpu.sync_copy(rows_v, out_hbm.at[wid])

    return k(q, mtop_flat)


def _attn_red(qred, k, v, lc):
    def body(q_ref, k_ref, v_ref, c_ref, vm_ref):
        s = lax.dot_general(q_ref[0][:, :D_K], k_ref[0][:, :D_K],
                            (((1,), (1,)), ((), ())),
                            preferred_element_type=jnp.float32) * (1.0 / math.sqrt(D_K))
        mx = jnp.max(s, axis=1, keepdims=True)
        e = jnp.exp(s - mx)
        attn = e / jnp.sum(e, axis=1, keepdims=True)
        c_ref[0] = jnp.dot(attn, v_ref[0], preferred_element_type=jnp.float32)
        vm_ref[0] = jnp.broadcast_to(jnp.mean(v_ref[0], axis=0, keepdims=True), (128, DP))

    return pl.pallas_call(
        body,
        grid=(BH,),
        in_specs=[pl.BlockSpec((1, UPAD, DP), lambda i: (i, 0, 0)),
                  pl.BlockSpec((1, lc, DP), lambda i: (i, 0, 0)),
                  pl.BlockSpec((1, lc, DP), lambda i: (i, 0, 0))],
        out_specs=[pl.BlockSpec((1, UPAD, DP), lambda i: (i, 0, 0)),
                   pl.BlockSpec((1, 128, DP), lambda i: (i, 0, 0))],
        out_shape=[jax.ShapeDtypeStruct((BH, UPAD, DP), jnp.float32),
                   jax.ShapeDtypeStruct((BH, 128, DP), jnp.float32)],
    )(qred, k, v)


def _sc_context(ctx, mtop_flat, vmrep, lc):
    mesh = plsc.VectorSubcoreMesh(core_axis_name="c", subcore_axis_name="s")
    nrep = 128

    @functools.partial(
        pl.kernel,
        out_type=jax.ShapeDtypeStruct((BH, lc, DP), jnp.float32),
        mesh=mesh,
        scratch_types=[pltpu.VMEM((UPAD,), jnp.int32),
                       pltpu.VMEM((UPAD, DP), jnp.float32),
                       pltpu.VMEM((nrep, DP), jnp.float32),
                       pltpu.SemaphoreType.DMA],
    )
    def k(ctx_hbm, mtop_hbm, vm_hbm, out_hbm, idx_v, rows_v, rep_v, sem):
        wid = lax.axis_index("s") * 2 + lax.axis_index("c")

        @pl.when(wid < BH)
        def _():
            pltpu.sync_copy(vm_hbm.at[wid], rep_v)
            for c0 in range(lc // nrep):
                pltpu.sync_copy(rep_v, out_hbm.at[wid, pl.ds(c0 * nrep, nrep)])
            pltpu.sync_copy(mtop_hbm.at[pl.ds(wid * UPAD, UPAD)], idx_v)
            pltpu.sync_copy(ctx_hbm.at[wid], rows_v)
            pltpu.async_copy(rows_v, out_hbm.at[wid].at[idx_v], sem).wait()

    return k(ctx, mtop_flat, vmrep)


def _wo_ln1(ctx, src, wo_r, bo, g, bln, lc):
    nt = lc // LT

    def body(c_ref, s_ref, w_ref, bo_ref, g_ref, b_ref, o_ref):
        acc = jnp.zeros((LT, D_MODEL), jnp.float32)
        for j in range(NHEAD // 4):
            z = jnp.concatenate([c_ref[4 * j + i][:, :D_K] for i in range(4)],
                                axis=1)
            acc = acc + jnp.dot(z, w_ref[j], preferred_element_type=jnp.float32)
        x = acc + bo_ref[...] + s_ref[...]
        o_ref[...] = _ln(x, g_ref[...], b_ref[...])

    return pl.pallas_call(
        body,
        grid=(B, nt),
        in_specs=[pl.BlockSpec((NHEAD, LT, DP), lambda b, t: (b, t, 0)),
                  pl.BlockSpec((LT, D_MODEL), lambda b, t: (b * nt + t, 0)),
                  pl.BlockSpec((NHEAD // 4, 4 * D_K, D_MODEL), lambda b, t: (0, 0, 0)),
                  pl.BlockSpec((1, D_MODEL), lambda b, t: (0, 0)),
                  pl.BlockSpec((1, D_MODEL), lambda b, t: (0, 0)),
                  pl.BlockSpec((1, D_MODEL), lambda b, t: (0, 0))],
        out_specs=pl.BlockSpec((LT, D_MODEL), lambda b, t: (b * nt + t, 0)),
        out_shape=jax.ShapeDtypeStruct((B * lc, D_MODEL), jnp.float32),
    )(ctx, src, wo_r, bo.reshape(1, -1), g.reshape(1, -1), bln.reshape(1, -1))


def _ffn_ln2(src, w1, b1, w2, b2, g, bln):
    n = src.shape[0]
    nt = n // LT

    def body(s_ref, w1_ref, b1_ref, w2_ref, b2_ref, g_ref, b_ref, o_ref):
        x = s_ref[...]
        t = jnp.dot(x, w1_ref[...], preferred_element_type=jnp.float32) + b1_ref[...]
        ge = 0.5 * t * (1.0 + lax.erf(t * (1.0 / math.sqrt(2.0))))
        y = jnp.dot(ge, w2_ref[...], preferred_element_type=jnp.float32) + b2_ref[...] + x
        o_ref[...] = _ln(y, g_ref[...], b_ref[...])

    return pl.pallas_call(
        body,
        grid=(nt,),
        in_specs=[pl.BlockSpec((LT, D_MODEL), lambda i: (i, 0)),
                  pl.BlockSpec((D_MODEL, DFF), lambda i: (0, 0)),
                  pl.BlockSpec((1, DFF), lambda i: (0, 0)),
                  pl.BlockSpec((DFF, D_MODEL), lambda i: (0, 0)),
                  pl.BlockSpec((1, D_MODEL), lambda i: (0, 0)),
                  pl.BlockSpec((1, D_MODEL), lambda i: (0, 0)),
                  pl.BlockSpec((1, D_MODEL), lambda i: (0, 0))],
        out_specs=pl.BlockSpec((LT, D_MODEL), lambda i: (i, 0)),
        out_shape=jax.ShapeDtypeStruct((n, D_MODEL), jnp.float32),
    )(src, w1, b1.reshape(1, -1), w2, b2.reshape(1, -1),
      g.reshape(1, -1), bln.reshape(1, -1))


def _conv_pool(src, w0, w1, w2, bc, scale, bnb, lc):
    def body(s_ref, w0_ref, w1_ref, w2_ref, bc_ref, sc_ref, bb_ref, o_ref):
        x = s_ref[...]
        xm = jnp.concatenate([x[lc - 1:], x[:lc - 1]], axis=0)
        xp = jnp.concatenate([x[1:], x[:1]], axis=0)
        y = (jnp.dot(xm, w0_ref[...], preferred_element_type=jnp.float32)
             + jnp.dot(x, w1_ref[...], preferred_element_type=jnp.float32)
             + jnp.dot(xp, w2_ref[...], preferred_element_type=jnp.float32)
             + bc_ref[...])
        y = y * sc_ref[...] + bb_ref[...]
        y = jnp.where(y > 0, y, jnp.exp(y) - 1.0)
        yr = y.reshape(lc // 2, 2, D_MODEL)
        a = yr[:, 0, :]
        bcol = yr[:, 1, :]
        bm = jnp.concatenate(
            [jnp.full((1, D_MODEL), -jnp.inf, jnp.float32), bcol[:-1]], axis=0)
        o_ref[...] = jnp.maximum(jnp.maximum(a, bcol), bm)

    return pl.pallas_call(
        body,
        grid=(B,),
        in_specs=[pl.BlockSpec((lc, D_MODEL), lambda b: (b, 0)),
                  pl.BlockSpec((D_MODEL, D_MODEL), lambda b: (0, 0)),
                  pl.BlockSpec((D_MODEL, D_MODEL), lambda b: (0, 0)),
                  pl.BlockSpec((D_MODEL, D_MODEL), lambda b: (0, 0)),
                  pl.BlockSpec((1, D_MODEL), lambda b: (0, 0)),
                  pl.BlockSpec((1, D_MODEL), lambda b: (0, 0)),
                  pl.BlockSpec((1, D_MODEL), lambda b: (0, 0))],
        out_specs=pl.BlockSpec((lc // 2, D_MODEL), lambda b: (b, 0)),
        out_shape=jax.ShapeDtypeStruct((B * (lc // 2), D_MODEL), jnp.float32),
    )(src, w0, w1, w2, bc.reshape(1, -1), scale.reshape(1, -1), bnb.reshape(1, -1))


def _gru(dec8, wi, wh, bi, bhb):
    def body(d_ref, wi_ref, wh_ref, bi_ref, bh_ref, hs_ref):
        gi = jnp.dot(d_ref[...], wi_ref[...], preferred_element_type=jnp.float32) + bi_ref[...]

        def step(t, h):
            gh = jnp.dot(h, wh_ref[...], preferred_element_type=jnp.float32) + bh_ref[...]
            r = jax.nn.sigmoid(gi[:, :DEC_H] + gh[:, :DEC_H])
            z = jax.nn.sigmoid(gi[:, DEC_H:2 * DEC_H] + gh[:, DEC_H:2 * DEC_H])
            nn = jnp.tanh(gi[:, 2 * DEC_H:] + r * gh[:, 2 * DEC_H:])
            hn = (1.0 - z) * nn + z * h
            hs_ref[pl.ds(t, 1)] = hn[None]
            return hn

        lax.fori_loop(0, PRED_H, step, jnp.zeros((8, DEC_H), jnp.float32))

    return pl.pallas_call(
        body,
        in_specs=[pl.BlockSpec((8, D_MODEL), lambda: (0, 0)),
                  pl.BlockSpec((D_MODEL, 3 * DEC_H), lambda: (0, 0)),
                  pl.BlockSpec((DEC_H, 3 * DEC_H), lambda: (0, 0)),
                  pl.BlockSpec((1, 3 * DEC_H), lambda: (0, 0)),
                  pl.BlockSpec((1, 3 * DEC_H), lambda: (0, 0))],
        out_specs=pl.BlockSpec((PRED_H, 8, DEC_H), lambda: (0, 0, 0)),
        out_shape=jax.ShapeDtypeStruct((PRED_H, 8, DEC_H), jnp.float32),
    )(dec8, wi, wh, bi.reshape(1, -1), bhb.reshape(1, -1))


def _outproj(hs_flat, w, b):
    n = hs_flat.shape[0]

    def body(h_ref, w_ref, b_ref, o_ref):
        o_ref[...] = jax.nn.sigmoid(
            jnp.dot(h_ref[...], w_ref[...], preferred_element_type=jnp.float32) + b_ref[...])

    return pl.pallas_call(
        body,
        in_specs=[pl.BlockSpec((n, DEC_H), lambda: (0, 0)),
                  pl.BlockSpec((DEC_H, 1), lambda: (0, 0)),
                  pl.BlockSpec((1, 1), lambda: (0, 0))],
        out_specs=pl.BlockSpec((n, 1), lambda: (0, 0)),
        out_shape=jax.ShapeDtypeStruct((n, 1), jnp.float32),
    )(hs_flat, w, b.reshape(1, 1))


def kernel(x, params):
    p = params
    x2 = x.reshape(B * L_IN, INPUT_DIM)
    pe = _pe_table(L_IN, D_MODEL)
    h = _embed(x2, p['emb_W'], p['emb_b'], pe)

    lc = L_IN
    for l in range(N_LAYERS):
        lp = p['layers'][l]
        u = min(FACTOR * int(np.ceil(np.log(lc + 1))), lc)
        cnt, msk = _sample_tables(l, lc, u)
        q, k, v = _qkv(h, lp['Wq'], lp['Wk'], lp['Wv'],
                       lp['bq'].reshape(1, -1), lp['bk'].reshape(1, -1),
                       lp['bv'].reshape(1, -1), lc)
        m = _m_scores(q, k, cnt, msk, lc)
        mtop = _topk(m, lc, u).reshape(BH * UPAD)
        qred = _sc_gather(q, mtop, lc)
        ctx, vmrep = _attn_red(qred, k, v, lc)
        ctxfull = _sc_context(ctx, mtop, vmrep, lc)
        wo_r = lp['Wo'].reshape(NHEAD // 4, 4 * D_K, D_MODEL)
        h = _wo_ln1(ctxfull, h, wo_r, lp['bo'], lp['ln1_g'], lp['ln1_b'], lc)
        h = _ffn_ln2(h, lp['W1'], lp['b1'], lp['W2'], lp['b2'],
                     lp['ln2_g'], lp['ln2_b'])
        if l < N_LAYERS - 1:
            cp = p['convs'][l]
            w0 = cp['w'][:, :, 0].T
            w1 = cp['w'][:, :, 1].T
            w2 = cp['w'][:, :, 2].T
            scale = cp['bn_g'] / jnp.sqrt(jnp.float32(1.0) + 1e-5)
            h = _conv_pool(h, w0, w1, w2, cp['b'], scale, cp['bn_b'], lc)
            lc = lc // 2

    dec_rows = jnp.stack([h[lc - 1], h[2 * lc - 1]], axis=0)
    dec8 = jnp.zeros((8, D_MODEL), jnp.float32).at[:B].set(dec_rows)
    hs = _gru(dec8, p['gru_Wi'], p['gru_Wh'], p['gru_bi'], p['gru_bh'])
    o = _outproj(hs.reshape(PRED_H * 8, DEC_H), p['out_W'], p['out_b'])
    return o.reshape(PRED_H, 8)[:, :B].T
